# Initial kernel scaffold; baseline (speedup 1.0000x reference)
import functools

import jax
import jax.numpy as jnp
from jax import lax
from jax.experimental import pallas as pl
from jax.experimental.pallas import tpu as pltpu
from jax.experimental.pallas import tpu_sc as plsc

EPS = 1e-5
G = 256
NUM_CHIRALITY = 11
CH = 128
NC = 2
NS = 16


def _mesh():
    return plsc.VectorSubcoreMesh(core_axis_name="c", subcore_axis_name="s")


@functools.lru_cache(maxsize=None)
def _make_k0(N, D, E):
    n_full = N // CH
    n_tail = N - n_full * CH
    ne = E // CH
    assert E % CH == 0 and N % NS == 0
    rpt = N // NS
    nw = NC * NS
    node_iters = -(-n_full // nw)
    edge_iters = -(-ne // nw)

    @functools.partial(
        pl.kernel,
        mesh=_mesh(),
        out_type=[
            jax.ShapeDtypeStruct((N, D), jnp.float32),
            jax.ShapeDtypeStruct((NC, N, 16), jnp.float32),
        ],
        scratch_types=[
            pltpu.VMEM((CH,), jnp.int32),
            pltpu.VMEM((CH,), jnp.int32),
            pltpu.VMEM((CH,), jnp.int32),
            pltpu.VMEM((16,), jnp.int32),
            pltpu.VMEM((CH, D), jnp.float32),
            pltpu.VMEM((CH,), jnp.int32),
            pltpu.VMEM((CH,), jnp.int32),
            pltpu.VMEM((CH,), jnp.int32),
            pltpu.VMEM((CH, 16), jnp.float32),
            pltpu.VMEM_SHARED((N, 16), jnp.float32),
            pltpu.SemaphoreType.DMA,
        ],
    )
    def k0(t_hbm, x0_hbm, x1_hbm, dst_hbm, ea0_hbm, ea1_hbm, z16_hbm,
           h0_hbm, cp_hbm,
           x0b, x1b, idxb, idxt, rows, dstb, ea0b, ea1b, ohb, cnt, sem):
        c = lax.axis_index("c")
        s = lax.axis_index("s")
        wid = s * NC + c

        pltpu.sync_copy(z16_hbm, cnt.at[pl.ds(s * rpt, rpt)])

        def zoh(i, carry):
            ohb[i, :] = jnp.zeros((16,), jnp.float32)
            return carry
        lax.fori_loop(0, CH, zoh, 0)

        def nbody(kk, carry):
            ch = kk * nw + wid

            @pl.when(ch < n_full)
            def _():
                base = ch * CH
                pltpu.sync_copy(x0_hbm.at[pl.ds(base, CH)], x0b)
                pltpu.sync_copy(x1_hbm.at[pl.ds(base, CH)], x1b)
                for r in range(CH // 16):
                    sl = pl.ds(r * 16, 16)
                    idxb[sl] = x0b[sl] * NUM_CHIRALITY + x1b[sl]
                pltpu.async_copy(t_hbm.at[idxb], rows, sem).wait()
                pltpu.sync_copy(rows, h0_hbm.at[pl.ds(base, CH), :])
            return carry
        lax.fori_loop(0, node_iters, nbody, 0)

        if n_tail:
            @pl.when(wid == 0)
            def _():
                base = n_full * CH
                pltpu.sync_copy(x0_hbm.at[pl.ds(base, n_tail)],
                                x0b.at[pl.ds(0, n_tail)])
                pltpu.sync_copy(x1_hbm.at[pl.ds(base, n_tail)],
                                x1b.at[pl.ds(0, n_tail)])
                for r in range(n_tail // 16):
                    sl = pl.ds(r * 16, 16)
                    idxt[sl] = x0b[sl] * NUM_CHIRALITY + x1b[sl]
                pltpu.async_copy(t_hbm.at[idxt], rows.at[pl.ds(0, n_tail)],
                                 sem).wait()
                pltpu.sync_copy(rows.at[pl.ds(0, n_tail)],
                                h0_hbm.at[pl.ds(base, n_tail), :])

        plsc.subcore_barrier()

        ones16 = jnp.full((16,), 1.0, jnp.float32)
        zeros16 = jnp.zeros((16,), jnp.float32)

        def ebody(kk, carry):
            ch = kk * nw + wid

            @pl.when(ch < ne)
            def _():
                base = ch * CH
                pltpu.sync_copy(dst_hbm.at[pl.ds(base, CH)], dstb)
                pltpu.sync_copy(ea0_hbm.at[pl.ds(base, CH)], ea0b)
                pltpu.sync_copy(ea1_hbm.at[pl.ds(base, CH)], ea1b)
                for r in range(CH // 16):
                    sl = pl.ds(r * 16, 16)
                    eid = lax.iota(jnp.int32, 16) + r * 16
                    plsc.store_scatter(ohb, [eid, ea0b[sl]], ones16)
                    plsc.store_scatter(ohb, [eid, ea1b[sl] + 8], ones16)
                pltpu.sync_copy(ohb, cnt.at[dstb], add=True)
                for r in range(CH // 16):
                    sl = pl.ds(r * 16, 16)
                    eid = lax.iota(jnp.int32, 16) + r * 16
                    plsc.store_scatter(ohb, [eid, ea0b[sl]], zeros16)
                    plsc.store_scatter(ohb, [eid, ea1b[sl] + 8], zeros16)
            return carry
        lax.fori_loop(0, edge_iters, ebody, 0)

        plsc.subcore_barrier()
        pltpu.sync_copy(cnt.at[pl.ds(s * rpt, rpt)],
                        cp_hbm.at[c, pl.ds(s * rpt, rpt), :])

    return k0


@functools.lru_cache(maxsize=None)
def _make_spmm(N, D, E):
    ne = E // CH
    rpt = N // NS
    nw = NC * NS
    edge_iters = -(-ne // nw)

    @functools.partial(
        pl.kernel,
        mesh=_mesh(),
        out_type=jax.ShapeDtypeStruct((NC, N, D), jnp.float32),
        scratch_types=[
            pltpu.VMEM((CH,), jnp.int32),
            pltpu.VMEM((CH,), jnp.int32),
            pltpu.VMEM((CH, D), jnp.float32),
            pltpu.VMEM_SHARED((N, D), jnp.float32),
            pltpu.SemaphoreType.DMA,
        ],
    )
    def spmm(h_hbm, src_hbm, dst_hbm, z_hbm, p_hbm, idxs, idxd, rows, acc, sem):
        c = lax.axis_index("c")
        s = lax.axis_index("s")
        wid = s * NC + c

        pltpu.sync_copy(z_hbm, acc.at[pl.ds(s * rpt, rpt)])
        plsc.subcore_barrier()

        def body(kk, carry):
            ch = kk * nw + wid

            @pl.when(ch < ne)
            def _():
                base = ch * CH
                pltpu.sync_copy(src_hbm.at[pl.ds(base, CH)], idxs)
                pltpu.async_copy(h_hbm.at[idxs], rows, sem).wait()
                pltpu.sync_copy(dst_hbm.at[pl.ds(base, CH)], idxd)
                pltpu.sync_copy(rows, acc.at[idxd], add=True)
            return carry
        lax.fori_loop(0, edge_iters, body, 0)

        plsc.subcore_barrier()
        pltpu.sync_copy(acc.at[pl.ds(s * rpt, rpt)],
                        p_hbm.at[c, pl.ds(s * rpt, rpt), :])

    return spmm


@functools.lru_cache(maxsize=None)
def _make_pool(N, D):
    n_full = N // CH
    n_tail = N - n_full * CH
    rpt = G // NS
    pool_iters = -(-n_full // NS)

    @functools.partial(
        pl.kernel,
        mesh=_mesh(),
        out_type=jax.ShapeDtypeStruct((G, D), jnp.float32),
        scratch_types=[
            pltpu.VMEM((CH, D), jnp.float32),
            pltpu.VMEM((CH,), jnp.int32),
            pltpu.VMEM((16,), jnp.int32),
            pltpu.VMEM((CH, 16), jnp.float32),
            pltpu.VMEM((rpt, D), jnp.float32),
            pltpu.VMEM((rpt, 16), jnp.float32),
            pltpu.VMEM_SHARED((G, D), jnp.float32),
            pltpu.VMEM_SHARED((G, 16), jnp.float32),
            pltpu.SemaphoreType.DMA,
        ],
    )
    def pool(h_hbm, batch_hbm, zg_hbm, zc_hbm, ga_hbm,
             rows, idxb, idxt, onesb, gv, cv, ssum, cnt, sem):
        c = lax.axis_index("c")
        s = lax.axis_index("s")

        pltpu.sync_copy(zg_hbm, ssum.at[pl.ds(s * rpt, rpt)])
        pltpu.sync_copy(zc_hbm, cnt.at[pl.ds(s * rpt, rpt)])

        def fill(i, carry):
            onesb[i, :] = jnp.full((16,), 1.0, jnp.float32)
            return carry
        lax.fori_loop(0, CH, fill, 0)
        plsc.subcore_barrier()

        @pl.when(c == 0)
        def _():
            def body(kk, carry):
                ch = kk * NS + s

                @pl.when(ch < n_full)
                def _():
                    base = ch * CH
                    pltpu.sync_copy(h_hbm.at[pl.ds(base, CH), :], rows)
                    pltpu.sync_copy(batch_hbm.at[pl.ds(base, CH)], idxb)
                    pltpu.sync_copy(rows, ssum.at[idxb], add=True)
                    pltpu.sync_copy(onesb, cnt.at[idxb], add=True)
                return carry
            lax.fori_loop(0, pool_iters, body, 0)

            if n_tail:
                @pl.when(s == 0)
                def _():
                    base = n_full * CH
                    pltpu.sync_copy(h_hbm.at[pl.ds(base, n_tail), :],
                                    rows.at[pl.ds(0, n_tail)])
                    pltpu.sync_copy(batch_hbm.at[pl.ds(base, n_tail)], idxt)
                    pltpu.sync_copy(rows.at[pl.ds(0, n_tail)],
                                    ssum.at[idxt], add=True)
                    pltpu.sync_copy(onesb.at[pl.ds(0, n_tail)],
                                    cnt.at[idxt], add=True)

        plsc.subcore_barrier()

        @pl.when(c == 0)
        def _():
            pltpu.sync_copy(ssum.at[pl.ds(s * rpt, rpt)], gv)
            pltpu.sync_copy(cnt.at[pl.ds(s * rpt, rpt)], cv)

            def div(i, carry):
                rec = 1.0 / jnp.maximum(cv[i, :], 1.0)
                for j in range(D // 16):
                    sl = pl.ds(j * 16, 16)
                    gv[i, sl] = gv[i, sl] * rec
                return carry
            lax.fori_loop(0, rpt, div, 0)
            pltpu.sync_copy(gv, ga_hbm.at[pl.ds(s * rpt, rpt), :])

    return pool


def _tc1_body(p0, p1, h, cc, ee, crow, w1, b1, w2, b2, o_hp, o_s1, o_s2):
    i = pl.program_id(0)
    agg = p0[...] + p1[...] + h[...] + crow[...]
    agg = agg + jnp.dot(cc[...], ee[...], preferred_element_type=jnp.float32)
    hid = jnp.maximum(
        jnp.dot(agg, w1[...], preferred_element_type=jnp.float32) + b1[...], 0.0)
    hp = jnp.dot(hid, w2[...], preferred_element_type=jnp.float32) + b2[...]
    o_hp[...] = hp

    @pl.when(i == 0)
    def _():
        o_s1[...] = jnp.zeros_like(o_s1)
        o_s2[...] = jnp.zeros_like(o_s2)

    o_s1[...] += jnp.sum(hp, axis=0, keepdims=True)
    o_s2[...] += jnp.sum(hp * hp, axis=0, keepdims=True)


def _tc1(p0, p1, h, cc, ee, crow, w1, b1, w2, b2):
    n, d = h.shape
    bn = 2000
    full = lambda shape: pl.BlockSpec(shape, lambda i: (0, 0))
    blk = lambda shape: pl.BlockSpec(shape, lambda i: (i, 0))
    return pl.pallas_call(
        _tc1_body,
        grid=(n // bn,),
        in_specs=[blk((bn, d)), blk((bn, d)), blk((bn, d)), blk((bn, 16)),
                  full((16, d)), full((1, d)), full((d, 2 * d)),
                  full((1, 2 * d)), full((2 * d, d)), full((1, d))],
        out_specs=[blk((bn, d)), full((1, d)), full((1, d))],
        out_shape=[jax.ShapeDtypeStruct((n, d), jnp.float32),
                   jax.ShapeDtypeStruct((1, d), jnp.float32),
                   jax.ShapeDtypeStruct((1, d), jnp.float32)],
    )(p0, p1, h, cc, ee, crow, w1, b1, w2, b2)


def _tc2_body(hp, s1, s2, g, b, o, *, n, relu):
    mean = s1[...] / n
    var = s2[...] / n - mean * mean
    inv = g[...] * lax.rsqrt(var + EPS)
    v = (hp[...] - mean) * inv + b[...]
    if relu:
        v = jnp.maximum(v, 0.0)
    o[...] = v


def _tc2(hp, s1, s2, g, b, relu):
    n, d = hp.shape
    bn = 2000
    full = lambda shape: pl.BlockSpec(shape, lambda i: (0, 0))
    blk = lambda shape: pl.BlockSpec(shape, lambda i: (i, 0))
    return pl.pallas_call(
        functools.partial(_tc2_body, n=float(n), relu=relu),
        grid=(n // bn,),
        in_specs=[blk((bn, d)), full((1, d)), full((1, d)), full((1, d)),
                  full((1, d))],
        out_specs=blk((bn, d)),
        out_shape=jax.ShapeDtypeStruct((n, d), jnp.float32),
    )(hp, s1, s2, g, b)


def kernel(x, edge_index, edge_attr, batch, xe1, xe2, ee1, ee2,
           W1, b1, W2, b2, gamma, beta):
    N = x.shape[0]
    D = xe1.shape[1]
    E = edge_index.shape[1]
    L = W1.shape[0]

    i32 = jnp.int32
    x0 = x[:, 0].astype(i32)
    x1 = x[:, 1].astype(i32)
    src = edge_index[0].astype(i32)
    dst = edge_index[1].astype(i32)
    ea0 = edge_attr[:, 0].astype(i32)
    ea1 = edge_attr[:, 1].astype(i32)
    bat = batch.astype(i32)

    T = (xe1[:, None, :] + xe2[None, :, :]).reshape(-1, D)

    z16 = jnp.zeros((N // NS, 16), jnp.float32)
    z128 = jnp.zeros((N // NS, D), jnp.float32)
    zg = jnp.zeros((G // NS, D), jnp.float32)
    zc = jnp.zeros((G // NS, 16), jnp.float32)

    k0 = _make_k0(N, D, E)
    spmm = _make_spmm(N, D, E)
    pool = _make_pool(N, D)

    h, cp = k0(T, x0, x1, dst, ea0, ea1, z16)
    cc = cp[0] + cp[1]

    for l in range(L):
        ee = jnp.zeros((16, D), jnp.float32)
        ee = ee.at[0:7].set(ee1[l]).at[8:11].set(ee2[l])
        crow = (ee1[l, 4] + ee2[l, 0]).reshape(1, D)
        p = spmm(h, src, dst, z128)
        hp, s1, s2 = _tc1(p[0], p[1], h, cc, ee, crow,
                          W1[l], b1[l].reshape(1, -1),
                          W2[l], b2[l].reshape(1, -1))
        h = _tc2(hp, s1, s2, gamma[l].reshape(1, -1), beta[l].reshape(1, -1),
                 relu=l < L - 1)

    ga = pool(h, bat, zg, zc)
    return ga, h


# speedup vs baseline: 1.0916x; 1.0916x over previous
import functools

import jax
import jax.numpy as jnp
from jax import lax
from jax.experimental import pallas as pl
from jax.experimental.pallas import tpu as pltpu
from jax.experimental.pallas import tpu_sc as plsc

EPS = 1e-5
G = 256
NUM_CHIRALITY = 11
CH = 128
NC = 2
NS = 16
BLK = 2000
_HI = jax.lax.Precision.HIGHEST


def _mesh():
    return plsc.VectorSubcoreMesh(core_axis_name="c", subcore_axis_name="s")


@functools.lru_cache(maxsize=None)
def _make_k0(N, D):
    n_full = N // CH
    n_tail = N - n_full * CH
    nw = NC * NS
    node_iters = -(-n_full // nw)

    @functools.partial(
        pl.kernel,
        mesh=_mesh(),
        out_type=jax.ShapeDtypeStruct((N, D), jnp.float32),
        scratch_types=[
            pltpu.VMEM((CH,), jnp.int32),
            pltpu.VMEM((CH,), jnp.int32),
            pltpu.VMEM((CH,), jnp.int32),
            pltpu.VMEM((16,), jnp.int32),
            pltpu.VMEM((CH, D), jnp.float32),
            pltpu.SemaphoreType.DMA,
        ],
    )
    def k0(t_hbm, x0_hbm, x1_hbm, h0_hbm, x0b, x1b, idxb, idxt, rows, sem):
        c = lax.axis_index("c")
        s = lax.axis_index("s")
        wid = s * NC + c

        def nbody(kk, carry):
            ch = kk * nw + wid

            @pl.when(ch < n_full)
            def _():
                base = ch * CH
                pltpu.sync_copy(x0_hbm.at[pl.ds(base, CH)], x0b)
                pltpu.sync_copy(x1_hbm.at[pl.ds(base, CH)], x1b)
                for r in range(CH // 16):
                    sl = pl.ds(r * 16, 16)
                    idxb[sl] = x0b[sl] * NUM_CHIRALITY + x1b[sl]
                pltpu.async_copy(t_hbm.at[idxb], rows, sem).wait()
                pltpu.sync_copy(rows, h0_hbm.at[pl.ds(base, CH), :])
            return carry
        lax.fori_loop(0, node_iters, nbody, 0)

        if n_tail:
            @pl.when(wid == 0)
            def _():
                base = n_full * CH
                pltpu.sync_copy(x0_hbm.at[pl.ds(base, n_tail)],
                                x0b.at[pl.ds(0, n_tail)])
                pltpu.sync_copy(x1_hbm.at[pl.ds(base, n_tail)],
                                x1b.at[pl.ds(0, n_tail)])
                for r in range(n_tail // 16):
                    sl = pl.ds(r * 16, 16)
                    idxt[sl] = x0b[sl] * NUM_CHIRALITY + x1b[sl]
                pltpu.async_copy(t_hbm.at[idxt], rows.at[pl.ds(0, n_tail)],
                                 sem).wait()
                pltpu.sync_copy(rows.at[pl.ds(0, n_tail)],
                                h0_hbm.at[pl.ds(base, n_tail), :])

    return k0


@functools.lru_cache(maxsize=None)
def _make_gather(N, D, E):
    ne = E // CH
    nw = NC * NS
    edge_iters = -(-ne // nw)

    @functools.partial(
        pl.kernel,
        mesh=_mesh(),
        out_type=jax.ShapeDtypeStruct((E, D), jnp.float32),
        scratch_types=[
            pltpu.VMEM((CH,), jnp.int32),
            pltpu.VMEM((CH, D), jnp.float32),
            pltpu.SemaphoreType.DMA,
        ],
    )
    def gat(h_hbm, src_hbm, hg_hbm, idxs, rows, sem):
        c = lax.axis_index("c")
        s = lax.axis_index("s")
        wid = s * NC + c

        def body(kk, carry):
            ch = kk * nw + wid

            @pl.when(ch < ne)
            def _():
                base = ch * CH
                pltpu.sync_copy(src_hbm.at[pl.ds(base, CH)], idxs)
                pltpu.async_copy(h_hbm.at[idxs], rows, sem).wait()
                pltpu.sync_copy(rows, hg_hbm.at[pl.ds(base, CH), :])
            return carry
        lax.fori_loop(0, edge_iters, body, 0)

    return gat


@functools.lru_cache(maxsize=None)
def _make_pool(N, D):
    n_full = N // CH
    n_tail = N - n_full * CH
    rpt = G // NS
    pool_iters = -(-n_full // NS)

    @functools.partial(
        pl.kernel,
        mesh=_mesh(),
        out_type=[jax.ShapeDtypeStruct((G, D), jnp.float32),
                  jax.ShapeDtypeStruct((G,), jnp.float32)],
        scratch_types=[
            pltpu.VMEM((CH, D), jnp.float32),
            pltpu.VMEM((CH,), jnp.int32),
            pltpu.VMEM((16,), jnp.int32),
            pltpu.VMEM((CH,), jnp.float32),
            pltpu.VMEM_SHARED((G, D), jnp.float32),
            pltpu.VMEM_SHARED((G,), jnp.float32),
            pltpu.SemaphoreType.DMA,
        ],
    )
    def pool(h_hbm, batch_hbm, zg_hbm, zc_hbm, ss_hbm, cnt_hbm,
             rows, idxb, idxt, onesb, ssum, cnt, sem):
        c = lax.axis_index("c")
        s = lax.axis_index("s")

        pltpu.sync_copy(zg_hbm, ssum.at[pl.ds(s * rpt, rpt)])

        @pl.when(s == 0)
        def _():
            pltpu.sync_copy(zc_hbm, cnt)

        for r in range(CH // 16):
            onesb[pl.ds(r * 16, 16)] = jnp.full((16,), 1.0, jnp.float32)
        plsc.subcore_barrier()

        @pl.when(c == 0)
        def _():
            def body(kk, carry):
                ch = kk * NS + s

                @pl.when(ch < n_full)
                def _():
                    base = ch * CH
                    pltpu.sync_copy(h_hbm.at[pl.ds(base, CH), :], rows)
                    pltpu.sync_copy(batch_hbm.at[pl.ds(base, CH)], idxb)
                    pltpu.sync_copy(rows, ssum.at[idxb], add=True)
                    pltpu.sync_copy(onesb, cnt.at[idxb], add=True)
                return carry
            lax.fori_loop(0, pool_iters, body, 0)

            if n_tail:
                @pl.when(s == 0)
                def _():
                    base = n_full * CH
                    pltpu.sync_copy(h_hbm.at[pl.ds(base, n_tail), :],
                                    rows.at[pl.ds(0, n_tail)])
                    pltpu.sync_copy(batch_hbm.at[pl.ds(base, n_tail)], idxt)
                    pltpu.sync_copy(rows.at[pl.ds(0, n_tail)],
                                    ssum.at[idxt], add=True)
                    pltpu.sync_copy(onesb.at[pl.ds(0, n_tail)],
                                    cnt.at[idxt], add=True)

        plsc.subcore_barrier()

        @pl.when(c == 0)
        def _():
            pltpu.sync_copy(ssum.at[pl.ds(s * rpt, rpt)],
                            ss_hbm.at[pl.ds(s * rpt, rpt), :])

            @pl.when(s == 0)
            def _():
                pltpu.sync_copy(cnt, cnt_hbm)

    return pool


def _scat_body(hg, combo, dst, e21, o, acc, msgs):
    i = pl.program_id(0)
    nsteps = pl.num_programs(0)

    @pl.when(i == 0)
    def _():
        acc[...] = jnp.zeros_like(acc)

    oh = (lax.broadcasted_iota(jnp.int32, (BLK, 32), 1) == combo[...]
          ).astype(jnp.float32)
    e = jnp.dot(oh, e21[...], precision=_HI,
                preferred_element_type=jnp.float32)
    msgs[...] = hg[...] + e

    def body(k, carry):
        d = dst[0, 0, k]
        acc[pl.ds(d, 1), :] += msgs[pl.ds(k, 1), :]
        return carry
    lax.fori_loop(0, BLK, body, 0)

    @pl.when(i == nsteps - 1)
    def _():
        o[...] = acc[...]


def _tc_scatter(hg, combo, dst2, e21, n):
    e, d = hg.shape
    return pl.pallas_call(
        _scat_body,
        grid=(e // BLK,),
        in_specs=[
            pl.BlockSpec((BLK, d), lambda i: (i, 0)),
            pl.BlockSpec((BLK, 1), lambda i: (i, 0)),
            pl.BlockSpec((1, 1, BLK), lambda i: (i, 0, 0),
                         memory_space=pltpu.SMEM),
            pl.BlockSpec((32, d), lambda i: (0, 0)),
        ],
        out_specs=pl.BlockSpec((n, d), lambda i: (0, 0)),
        out_shape=jax.ShapeDtypeStruct((n, d), jnp.float32),
        scratch_shapes=[pltpu.VMEM((n, d), jnp.float32),
                        pltpu.VMEM((BLK, d), jnp.float32)],
    )(hg, combo, dst2, e21)


def _tree8(acc):
    a4 = acc[0:4, :] + acc[4:8, :]
    a2 = a4[0:2, :] + a4[2:4, :]
    return a2[0:1, :] + a2[1:2, :]


def _tc1_body(p, h, crow, w1, b1, w2, b2, o_hp, o_s1, acc):
    i = pl.program_id(0)
    nsteps = pl.num_programs(0)
    agg = p[...] + (h[...] + crow[...])
    hid = jnp.maximum(
        jnp.dot(agg, w1[...], preferred_element_type=jnp.float32) + b1[...], 0.0)
    o_hp[...] = jnp.dot(hid, w2[...],
                        preferred_element_type=jnp.float32) + b2[...]

    @pl.when(i == 0)
    def _():
        acc[...] = jnp.zeros_like(acc)

    bn = o_hp.shape[0]

    def body(g, carry):
        acc[...] += o_hp[pl.ds(g * 8, 8), :]
        return carry
    lax.fori_loop(0, bn // 8, body, 0)

    @pl.when(i == nsteps - 1)
    def _():
        o_s1[...] = _tree8(acc[...])


def _tc1(p, h, crow, w1, b1, w2, b2):
    n, d = h.shape
    bn = 2000
    full = lambda shape: pl.BlockSpec(shape, lambda i: (0, 0))
    blk = lambda shape: pl.BlockSpec(shape, lambda i: (i, 0))
    return pl.pallas_call(
        _tc1_body,
        grid=(n // bn,),
        in_specs=[blk((bn, d)), blk((bn, d)), full((1, d)),
                  full((d, 2 * d)), full((1, 2 * d)), full((2 * d, d)),
                  full((1, d))],
        out_specs=[blk((bn, d)), full((1, d))],
        out_shape=[jax.ShapeDtypeStruct((n, d), jnp.float32),
                   jax.ShapeDtypeStruct((1, d), jnp.float32)],
        scratch_shapes=[pltpu.VMEM((8, d), jnp.float32)],
    )(p, h, crow, w1, b1, w2, b2)


def _tcv_body(hp, mean, o_s2, acc):
    i = pl.program_id(0)
    nsteps = pl.num_programs(0)

    @pl.when(i == 0)
    def _():
        acc[...] = jnp.zeros_like(acc)

    bn = hp.shape[0]

    def body(g, carry):
        dev = hp[pl.ds(g * 8, 8), :] - mean[...]
        acc[...] += dev * dev
        return carry
    lax.fori_loop(0, bn // 8, body, 0)

    @pl.when(i == nsteps - 1)
    def _():
        o_s2[...] = _tree8(acc[...])


def _tcv(hp, mean):
    n, d = hp.shape
    bn = 2000
    full = lambda shape: pl.BlockSpec(shape, lambda i: (0, 0))
    blk = lambda shape: pl.BlockSpec(shape, lambda i: (i, 0))
    return pl.pallas_call(
        _tcv_body,
        grid=(n // bn,),
        in_specs=[blk((bn, d)), full((1, d))],
        out_specs=full((1, d)),
        out_shape=jax.ShapeDtypeStruct((1, d), jnp.float32),
        scratch_shapes=[pltpu.VMEM((8, d), jnp.float32)],
    )(hp, mean)


def _tc2_body(hp, mean, var, g, b, o, *, relu):
    v = (hp[...] - mean[...]) / jnp.sqrt(var[...] + EPS) * g[...] + b[...]
    if relu:
        v = jnp.maximum(v, 0.0)
    o[...] = v


def _tc2(hp, mean, var, g, b, relu):
    n, d = hp.shape
    bn = 2000
    full = lambda shape: pl.BlockSpec(shape, lambda i: (0, 0))
    blk = lambda shape: pl.BlockSpec(shape, lambda i: (i, 0))
    return pl.pallas_call(
        functools.partial(_tc2_body, relu=relu),
        grid=(n // bn,),
        in_specs=[blk((bn, d)), full((1, d)), full((1, d)), full((1, d)),
                  full((1, d))],
        out_specs=blk((bn, d)),
        out_shape=jax.ShapeDtypeStruct((n, d), jnp.float32),
    )(hp, mean, var, g, b)


def _tc3_body(ss, cnt, o):
    o[...] = ss[...] / jnp.clip(cnt[...], 1.0, None)


def _tc3(ss, cnt):
    g, d = ss.shape
    return pl.pallas_call(
        _tc3_body,
        in_specs=[pl.BlockSpec((g, d), lambda: (0, 0)),
                  pl.BlockSpec((g, 1), lambda: (0, 0))],
        out_specs=pl.BlockSpec((g, d), lambda: (0, 0)),
        out_shape=jax.ShapeDtypeStruct((g, d), jnp.float32),
    )(ss, cnt)


def kernel(x, edge_index, edge_attr, batch, xe1, xe2, ee1, ee2,
           W1, b1, W2, b2, gamma, beta):
    N = x.shape[0]
    D = xe1.shape[1]
    E = edge_index.shape[1]
    L = W1.shape[0]

    i32 = jnp.int32
    x0 = x[:, 0].astype(i32)
    x1 = x[:, 1].astype(i32)
    src = edge_index[0].astype(i32)
    dst2 = edge_index[1].astype(i32).reshape(E // BLK, 1, BLK)
    combo = (edge_attr[:, 0].astype(i32) * 3
             + edge_attr[:, 1].astype(i32)).reshape(E, 1)
    bat = batch.astype(i32)

    T = (xe1[:, None, :] + xe2[None, :, :]).reshape(-1, D)

    zg = jnp.zeros((G // NS, D), jnp.float32)
    zc = jnp.zeros((G,), jnp.float32)

    k0 = _make_k0(N, D)
    gat = _make_gather(N, D, E)
    pool = _make_pool(N, D)

    h = k0(T, x0, x1)

    for l in range(L):
        e21 = jnp.zeros((32, D), jnp.float32)
        e21 = e21.at[0:21].set(
            (ee1[l][:, None, :] + ee2[l][None, :, :]).reshape(21, D))
        crow = (ee1[l, 4] + ee2[l, 0]).reshape(1, D)
        hg = gat(h, src)
        p = _tc_scatter(hg, combo, dst2, e21, N)
        hp, s1 = _tc1(p, h, crow, W1[l], b1[l].reshape(1, -1),
                      W2[l], b2[l].reshape(1, -1))
        mean = s1 / jnp.float32(N)
        var = _tcv(hp, mean) / jnp.float32(N)
        h = _tc2(hp, mean, var, gamma[l].reshape(1, -1),
                 beta[l].reshape(1, -1), relu=l < L - 1)

    ss, cnt = pool(h, bat, zg, zc)
    ga = _tc3(ss, cnt.reshape(G, 1))
    return ga, h
